# initial kernel scaffold (unmeasured)
import jax
import jax.numpy as jnp
from jax import lax
from jax.experimental import pallas as pl
from jax.experimental.pallas import tpu as pltpu

N_DEV = 4
SQ = 2048
SKV = 2048
DH = 128
SCALE = 0.08838834764831843
QBLK = 512


def kernel(x, Wq, K_ext, V_ext, Wo):
    my = lax.axis_index("i")
    _, Sq, Dm = x.shape
    _, Skv, Hloc, Dh = K_ext.shape
    dloc = Hloc * Dh
    Dout = Wo.shape[1]

    x2 = x[0].astype(jnp.bfloat16)
    Wq_my = lax.dynamic_slice(Wq, (0, my * dloc), (Dm, dloc)).astype(jnp.bfloat16)
    K2 = K_ext[0].reshape(Skv, dloc).astype(jnp.bfloat16)
    V2 = V_ext[0].reshape(Skv, dloc).astype(jnp.bfloat16)
    Wo_my = lax.dynamic_slice(Wo, (my * dloc, 0), (dloc, Dout)).astype(jnp.bfloat16)

    def body(x_ref, wq_ref, k_ref, v_ref, wo_ref, out_ref,
             comm_ref, send_sems, recv_sems):
        my_pos = lax.axis_index("i")
        left = lax.rem(my_pos + N_DEV - 1, N_DEV)
        right = lax.rem(my_pos + 1, N_DEV)

        q = lax.dot_general(
            x_ref[...], wq_ref[...], (((1,), (0,)), ((), ())),
            preferred_element_type=jnp.bfloat16)

        for qb in range(SQ // QBLK):
            acc = jnp.zeros((QBLK, Dout), jnp.float32)
            for h in range(Hloc):
                qh = q[qb * QBLK:(qb + 1) * QBLK, h * DH:(h + 1) * DH]
                kh = k_ref[:, h * DH:(h + 1) * DH]
                s = lax.dot_general(
                    qh, kh, (((1,), (1,)), ((), ())),
                    preferred_element_type=jnp.float32) * SCALE
                qi = lax.broadcasted_iota(jnp.int32, (QBLK, SKV), 0) + qb * QBLK
                ki = lax.broadcasted_iota(jnp.int32, (QBLK, SKV), 1)
                mask = (jnp.abs(qi - ki) <= 128) | (ki < 32) | (qi < 32)
                s = jnp.where(mask, s, -1e9)
                m = jnp.max(s, axis=1, keepdims=True)
                w = jnp.exp(s - m)
                denom = jnp.sum(w, axis=1, keepdims=True)
                wb = (w / denom).astype(jnp.bfloat16)
                ctx = lax.dot_general(
                    wb, v_ref[:, h * DH:(h + 1) * DH],
                    (((1,), (0,)), ((), ())),
                    preferred_element_type=jnp.bfloat16)
                acc = acc + lax.dot_general(
                    ctx, wo_ref[h * DH:(h + 1) * DH, :],
                    (((1,), (0,)), ((), ())),
                    preferred_element_type=jnp.float32)
            out_ref[qb * QBLK:(qb + 1) * QBLK, :] = acc
            comm_ref[0, qb * QBLK:(qb + 1) * QBLK, :] = acc.astype(jnp.bfloat16)

        barrier_sem = pltpu.get_barrier_semaphore()
        for nbr in [left, right]:
            pl.semaphore_signal(barrier_sem, inc=1, device_id=(nbr,),
                                device_id_type=pl.DeviceIdType.MESH)
        pl.semaphore_wait(barrier_sem, 2)

        for hop in range(N_DEV - 1):
            rdma = pltpu.make_async_remote_copy(
                src_ref=comm_ref.at[hop],
                dst_ref=comm_ref.at[hop + 1],
                send_sem=send_sems.at[hop],
                recv_sem=recv_sems.at[hop],
                device_id=(right,),
                device_id_type=pl.DeviceIdType.MESH,
            )
            rdma.start()
            rdma.wait()
            out_ref[...] = out_ref[...] + comm_ref[hop + 1].astype(jnp.float32)

    out = pl.pallas_call(
        body,
        out_shape=jax.ShapeDtypeStruct((Sq, Dout), jnp.float32),
        in_specs=[pl.BlockSpec(memory_space=pltpu.VMEM)] * 5,
        out_specs=pl.BlockSpec(memory_space=pltpu.VMEM),
        scratch_shapes=[
            pltpu.VMEM((N_DEV, SQ, Dout), jnp.bfloat16),
            pltpu.SemaphoreType.DMA((N_DEV - 1,)),
            pltpu.SemaphoreType.DMA((N_DEV - 1,)),
        ],
        compiler_params=pltpu.CompilerParams(collective_id=0),
    )(x2, Wq_my, K2, V2, Wo_my)
    return out[None]


# baseline (device time: 281546 ns/iter reference)
import jax
import jax.numpy as jnp
from jax import lax
from jax.experimental import pallas as pl
from jax.experimental.pallas import tpu as pltpu

N_DEV = 4
SQ = 2048
SKV = 2048
DH = 128
SCALE = 0.08838834764831843
QBLK = 512


def kernel(x, Wq, K_ext, V_ext, Wo):
    my = lax.axis_index("i")
    _, Sq, Dm = x.shape
    _, Skv, Hloc, Dh = K_ext.shape
    dloc = Hloc * Dh
    Dout = Wo.shape[1]

    x2 = x[0].astype(jnp.bfloat16)
    Wq_my = lax.dynamic_slice(Wq, (0, my * dloc), (Dm, dloc)).astype(jnp.bfloat16)
    K2 = K_ext[0].reshape(Skv, dloc).astype(jnp.bfloat16)
    V2 = V_ext[0].reshape(Skv, dloc).astype(jnp.bfloat16)
    Wo_my = lax.dynamic_slice(Wo, (my * dloc, 0), (dloc, Dout)).astype(jnp.bfloat16)

    def body(x_ref, wq_ref, k_ref, v_ref, wo_ref, out_ref,
             comm_ref, send_sems, recv_sems):
        my_pos = lax.axis_index("i")
        left = lax.rem(my_pos + N_DEV - 1, N_DEV)
        right = lax.rem(my_pos + 1, N_DEV)

        q = lax.dot_general(
            x_ref[...], wq_ref[...], (((1,), (0,)), ((), ())),
            preferred_element_type=jnp.float32).astype(jnp.bfloat16)

        for qb in range(SQ // QBLK):
            acc = jnp.zeros((QBLK, Dout), jnp.float32)
            for h in range(Hloc):
                qh = q[qb * QBLK:(qb + 1) * QBLK, h * DH:(h + 1) * DH]
                kh = k_ref[:, h * DH:(h + 1) * DH]
                s = lax.dot_general(
                    qh, kh, (((1,), (1,)), ((), ())),
                    preferred_element_type=jnp.float32) * SCALE
                qi = lax.broadcasted_iota(jnp.int32, (QBLK, SKV), 0) + qb * QBLK
                ki = lax.broadcasted_iota(jnp.int32, (QBLK, SKV), 1)
                mask = (jnp.abs(qi - ki) <= 128) | (ki < 32) | (qi < 32)
                s = jnp.where(mask, s, -1e9)
                m = jnp.max(s, axis=1, keepdims=True)
                w = jnp.exp(s - m)
                denom = jnp.sum(w, axis=1, keepdims=True)
                wb = (w / denom).astype(jnp.bfloat16)
                ctx = lax.dot_general(
                    wb, v_ref[:, h * DH:(h + 1) * DH],
                    (((1,), (0,)), ((), ())),
                    preferred_element_type=jnp.float32).astype(jnp.bfloat16)
                acc = acc + lax.dot_general(
                    ctx, wo_ref[h * DH:(h + 1) * DH, :],
                    (((1,), (0,)), ((), ())),
                    preferred_element_type=jnp.float32)
            out_ref[qb * QBLK:(qb + 1) * QBLK, :] = acc
            comm_ref[0, qb * QBLK:(qb + 1) * QBLK, :] = acc.astype(jnp.bfloat16)

        barrier_sem = pltpu.get_barrier_semaphore()
        for nbr in [left, right]:
            pl.semaphore_signal(barrier_sem, inc=1, device_id=(nbr,),
                                device_id_type=pl.DeviceIdType.MESH)
        pl.semaphore_wait(barrier_sem, 2)

        for hop in range(N_DEV - 1):
            rdma = pltpu.make_async_remote_copy(
                src_ref=comm_ref.at[hop],
                dst_ref=comm_ref.at[hop + 1],
                send_sem=send_sems.at[hop],
                recv_sem=recv_sems.at[hop],
                device_id=(right,),
                device_id_type=pl.DeviceIdType.MESH,
            )
            rdma.start()
            rdma.wait()
            out_ref[...] = out_ref[...] + comm_ref[hop + 1].astype(jnp.float32)

    out = pl.pallas_call(
        body,
        out_shape=jax.ShapeDtypeStruct((Sq, Dout), jnp.float32),
        in_specs=[pl.BlockSpec(memory_space=pltpu.VMEM)] * 5,
        out_specs=pl.BlockSpec(memory_space=pltpu.VMEM),
        scratch_shapes=[
            pltpu.VMEM((N_DEV, SQ, Dout), jnp.bfloat16),
            pltpu.SemaphoreType.DMA((N_DEV - 1,)),
            pltpu.SemaphoreType.DMA((N_DEV - 1,)),
        ],
        compiler_params=pltpu.CompilerParams(
            collective_id=0, vmem_limit_bytes=100 * 1024 * 1024),
    )(x2, Wq_my, K2, V2, Wo_my)
    return out[None]


# device time: 198331 ns/iter; 1.4196x vs baseline; 1.4196x over previous
import jax
import jax.numpy as jnp
from jax import lax
from jax.experimental import pallas as pl
from jax.experimental.pallas import tpu as pltpu

N_DEV = 4
SQ = 2048
SKV = 2048
DH = 128
SCALE = 0.08838834764831843
QBLK = 512
N_CHUNK = SQ // QBLK


def kernel(x, Wq, K_ext, V_ext, Wo):
    my = lax.axis_index("i")
    _, Sq, Dm = x.shape
    _, Skv, Hloc, Dh = K_ext.shape
    dloc = Hloc * Dh
    Dout = Wo.shape[1]

    x2 = x[0].astype(jnp.bfloat16)
    Wq_my = lax.dynamic_slice(Wq, (0, my * dloc), (Dm, dloc)).astype(jnp.bfloat16)
    K2 = K_ext[0].reshape(Skv, dloc).astype(jnp.bfloat16)
    V2 = V_ext[0].reshape(Skv, dloc).astype(jnp.bfloat16)
    Wo_my = lax.dynamic_slice(Wo, (my * dloc, 0), (dloc, Dout)).astype(jnp.bfloat16)

    def body(x_ref, wq_ref, k_ref, v_ref, wo_ref, out_ref,
             q_scr, rs_send, rs_recv, ag_buf,
             rs_send_sems, rs_recv_sems, ag_send_sems, ag_recv_sems):
        my_pos = lax.axis_index("i")
        left = lax.rem(my_pos + N_DEV - 1, N_DEV)
        right = lax.rem(my_pos + 1, N_DEV)

        barrier_sem = pltpu.get_barrier_semaphore()
        for nbr in [left, right]:
            pl.semaphore_signal(barrier_sem, inc=1, device_id=(nbr,),
                                device_id_type=pl.DeviceIdType.MESH)
        pl.semaphore_wait(barrier_sem, 2)

        q_scr[...] = lax.dot_general(
            x_ref[...], wq_ref[...], (((1,), (0,)), ((), ())),
            preferred_element_type=jnp.float32).astype(jnp.bfloat16)

        def compute_chunk(c):
            qblk = q_scr[pl.ds(c * QBLK, QBLK), :]
            acc = jnp.zeros((QBLK, Dout), jnp.float32)
            qi = lax.broadcasted_iota(jnp.int32, (QBLK, SKV), 0) + c * QBLK
            ki = lax.broadcasted_iota(jnp.int32, (QBLK, SKV), 1)
            mask = (jnp.abs(qi - ki) <= 128) | (ki < 32) | (qi < 32)
            for h in range(Hloc):
                qh = qblk[:, h * DH:(h + 1) * DH]
                kh = k_ref[:, h * DH:(h + 1) * DH]
                s = lax.dot_general(
                    qh, kh, (((1,), (1,)), ((), ())),
                    preferred_element_type=jnp.float32) * SCALE
                s = jnp.where(mask, s, -1e9)
                m = jnp.max(s, axis=1, keepdims=True)
                w = jnp.exp(s - m)
                denom = jnp.sum(w, axis=1, keepdims=True)
                wb = (w / denom).astype(jnp.bfloat16)
                ctx = lax.dot_general(
                    wb, v_ref[:, h * DH:(h + 1) * DH],
                    (((1,), (0,)), ((), ())),
                    preferred_element_type=jnp.float32).astype(jnp.bfloat16)
                acc = acc + lax.dot_general(
                    ctx, wo_ref[h * DH:(h + 1) * DH, :],
                    (((1,), (0,)), ((), ())),
                    preferred_element_type=jnp.float32)
            return acc

        def rs_rdma(s):
            return pltpu.make_async_remote_copy(
                src_ref=rs_send.at[s], dst_ref=rs_recv.at[s],
                send_sem=rs_send_sems.at[s], recv_sem=rs_recv_sems.at[s],
                device_id=(right,), device_id_type=pl.DeviceIdType.MESH)

        acc = compute_chunk(my_pos)
        rs_send[0] = acc.astype(jnp.bfloat16)
        rdma = rs_rdma(0)
        rdma.start()
        for s in range(1, N_DEV - 1):
            c = lax.rem(my_pos + N_DEV - s, N_DEV)
            acc = compute_chunk(c)
            rdma.wait()
            red = acc + rs_recv[s - 1].astype(jnp.float32)
            rs_send[s] = red.astype(jnp.bfloat16)
            rdma = rs_rdma(s)
            rdma.start()
        c_own = lax.rem(my_pos + 1, N_DEV)
        acc = compute_chunk(c_own)
        rdma.wait()
        final = acc + rs_recv[N_DEV - 2].astype(jnp.float32)
        out_ref[pl.ds(c_own * QBLK, QBLK), :] = final

        ag_buf[0] = final.astype(jnp.bfloat16)
        for t in range(N_DEV - 1):
            rdma = pltpu.make_async_remote_copy(
                src_ref=ag_buf.at[t], dst_ref=ag_buf.at[t + 1],
                send_sem=ag_send_sems.at[t], recv_sem=ag_recv_sems.at[t],
                device_id=(right,), device_id_type=pl.DeviceIdType.MESH)
            rdma.start()
            rdma.wait()
            c_in = lax.rem(my_pos + N_DEV - t, N_DEV)
            out_ref[pl.ds(c_in * QBLK, QBLK), :] = (
                ag_buf[t + 1].astype(jnp.float32))

    out = pl.pallas_call(
        body,
        out_shape=jax.ShapeDtypeStruct((Sq, Dout), jnp.float32),
        in_specs=[pl.BlockSpec(memory_space=pltpu.VMEM)] * 5,
        out_specs=pl.BlockSpec(memory_space=pltpu.VMEM),
        scratch_shapes=[
            pltpu.VMEM((SQ, dloc), jnp.bfloat16),
            pltpu.VMEM((N_DEV - 1, QBLK, Dout), jnp.bfloat16),
            pltpu.VMEM((N_DEV - 1, QBLK, Dout), jnp.bfloat16),
            pltpu.VMEM((N_DEV, QBLK, Dout), jnp.bfloat16),
            pltpu.SemaphoreType.DMA((N_DEV - 1,)),
            pltpu.SemaphoreType.DMA((N_DEV - 1,)),
            pltpu.SemaphoreType.DMA((N_DEV - 1,)),
            pltpu.SemaphoreType.DMA((N_DEV - 1,)),
        ],
        compiler_params=pltpu.CompilerParams(
            collective_id=0, vmem_limit_bytes=100 * 1024 * 1024),
    )(x2, Wq_my, K2, V2, Wo_my)
    return out[None]


# device time: 150516 ns/iter; 1.8705x vs baseline; 1.3177x over previous
import jax
import jax.numpy as jnp
from jax import lax
from jax.experimental import pallas as pl
from jax.experimental.pallas import tpu as pltpu

N_DEV = 4
SQ = 2048
SKV = 2048
DH = 128
SCALE = 0.08838834764831843
QBLK = 512
N_CHUNK = SQ // QBLK


def kernel(x, Wq, K_ext, V_ext, Wo):
    my = lax.axis_index("i")
    _, Sq, Dm = x.shape
    _, Skv, Hloc, Dh = K_ext.shape
    dloc = Hloc * Dh
    Dout = Wo.shape[1]

    x2 = x[0].astype(jnp.bfloat16)
    Wq_my = lax.dynamic_slice(Wq, (0, my * dloc), (Dm, dloc)).astype(jnp.bfloat16)
    K2 = K_ext[0].reshape(Skv, dloc).astype(jnp.bfloat16)
    V2 = V_ext[0].reshape(Skv, dloc).astype(jnp.bfloat16)
    Wo_my = lax.dynamic_slice(Wo, (my * dloc, 0), (dloc, Dout)).astype(jnp.bfloat16)

    def body(x_ref, wq_ref, k_ref, v_ref, wo_ref, out_ref,
             q_scr, rs_send, rs_recv, ag_buf,
             rs_send_sems, rs_recv_sems, ag_send_sems, ag_recv_sems):
        my_pos = lax.axis_index("i")
        left = lax.rem(my_pos + N_DEV - 1, N_DEV)
        right = lax.rem(my_pos + 1, N_DEV)

        barrier_sem = pltpu.get_barrier_semaphore()
        for nbr in [left, right]:
            pl.semaphore_signal(barrier_sem, inc=1, device_id=(nbr,),
                                device_id_type=pl.DeviceIdType.MESH)
        pl.semaphore_wait(barrier_sem, 2)

        q_scr[...] = lax.dot_general(
            x_ref[...], wq_ref[...], (((1,), (0,)), ((), ())),
            preferred_element_type=jnp.float32).astype(jnp.bfloat16)

        BAND = 896
        GW = 128
        CW = SKV - BAND
        TR = 64

        def dotf32(a, b, dims):
            return lax.dot_general(a, b, (dims, ((), ())),
                                   preferred_element_type=jnp.float32)

        def compute_chunk(c):
            cs = c * QBLK
            bs = pl.multiple_of(jnp.clip(cs - 128, 0, SKV - BAND), 128)
            qblk = q_scr[pl.ds(cs, QBLK), :]
            kB = k_ref[pl.ds(bs, BAND), :]
            vB = v_ref[pl.ds(bs, BAND), :]

            qiA = lax.broadcasted_iota(jnp.int32, (QBLK, GW), 0) + cs
            kiA = lax.broadcasted_iota(jnp.int32, (QBLK, GW), 1)
            mA = (jnp.abs(qiA - kiA) <= 128) | (kiA < 32) | (qiA < 32)
            mA = mA & (kiA < bs)
            qiB = lax.broadcasted_iota(jnp.int32, (QBLK, BAND), 0) + cs
            kiB = lax.broadcasted_iota(jnp.int32, (QBLK, BAND), 1) + bs
            mB = (jnp.abs(qiB - kiB) <= 128) | (kiB < 32) | (qiB < 32)
            qiC = lax.broadcasted_iota(jnp.int32, (TR, CW), 0) + cs
            kiC = lax.broadcasted_iota(jnp.int32, (TR, CW), 1) + BAND
            mC = (qiC < 32) & (kiC >= bs + BAND)

            acc = jnp.zeros((QBLK, Dout), jnp.float32)
            for h in range(Hloc):
                hd = slice(h * DH, (h + 1) * DH)
                qh = qblk[:, hd]
                sA = dotf32(qh, k_ref[0:GW, hd], ((1,), (1,))) * SCALE
                sB = dotf32(qh, kB[:, hd], ((1,), (1,))) * SCALE
                sC = dotf32(qh[0:TR, :], k_ref[BAND:, hd],
                            ((1,), (1,))) * SCALE
                sA = jnp.where(mA, sA, -1e9)
                sB = jnp.where(mB, sB, -1e9)
                sC = jnp.where(mC, sC, -1e9)

                m_t = jnp.maximum(
                    jnp.maximum(jnp.max(sA[0:TR], axis=1, keepdims=True),
                                jnp.max(sB[0:TR], axis=1, keepdims=True)),
                    jnp.max(sC, axis=1, keepdims=True))
                wA_t = jnp.exp(sA[0:TR] - m_t)
                wB_t = jnp.exp(sB[0:TR] - m_t)
                wC_t = jnp.exp(sC - m_t)
                den_t = (jnp.sum(wA_t, axis=1, keepdims=True)
                         + jnp.sum(wB_t, axis=1, keepdims=True)
                         + jnp.sum(wC_t, axis=1, keepdims=True))
                ctx_t = (dotf32(wA_t.astype(jnp.bfloat16),
                                v_ref[0:GW, hd], ((1,), (0,)))
                         + dotf32(wB_t.astype(jnp.bfloat16),
                                  vB[:, hd], ((1,), (0,)))
                         + dotf32(wC_t.astype(jnp.bfloat16),
                                  v_ref[BAND:, hd], ((1,), (0,)))) / den_t

                m_b = jnp.maximum(
                    jnp.max(sA[TR:], axis=1, keepdims=True),
                    jnp.max(sB[TR:], axis=1, keepdims=True))
                wA_b = jnp.exp(sA[TR:] - m_b)
                wB_b = jnp.exp(sB[TR:] - m_b)
                den_b = (jnp.sum(wA_b, axis=1, keepdims=True)
                         + jnp.sum(wB_b, axis=1, keepdims=True))
                ctx_b = (dotf32(wA_b.astype(jnp.bfloat16),
                                v_ref[0:GW, hd], ((1,), (0,)))
                         + dotf32(wB_b.astype(jnp.bfloat16),
                                  vB[:, hd], ((1,), (0,)))) / den_b

                ctx = jnp.concatenate([ctx_t, ctx_b], axis=0)
                acc = acc + dotf32(ctx.astype(jnp.bfloat16),
                                   wo_ref[hd, :], ((1,), (0,)))
            return acc

        def rs_rdma(s):
            return pltpu.make_async_remote_copy(
                src_ref=rs_send.at[s], dst_ref=rs_recv.at[s],
                send_sem=rs_send_sems.at[s], recv_sem=rs_recv_sems.at[s],
                device_id=(right,), device_id_type=pl.DeviceIdType.MESH)

        acc = compute_chunk(my_pos)
        rs_send[0] = acc.astype(jnp.bfloat16)
        rdma = rs_rdma(0)
        rdma.start()
        for s in range(1, N_DEV - 1):
            c = lax.rem(my_pos + N_DEV - s, N_DEV)
            acc = compute_chunk(c)
            rdma.wait()
            red = acc + rs_recv[s - 1].astype(jnp.float32)
            rs_send[s] = red.astype(jnp.bfloat16)
            rdma = rs_rdma(s)
            rdma.start()
        c_own = lax.rem(my_pos + 1, N_DEV)
        acc = compute_chunk(c_own)
        rdma.wait()
        final = acc + rs_recv[N_DEV - 2].astype(jnp.float32)
        out_ref[pl.ds(c_own * QBLK, QBLK), :] = final

        ag_buf[0] = final.astype(jnp.bfloat16)
        for t in range(N_DEV - 1):
            rdma = pltpu.make_async_remote_copy(
                src_ref=ag_buf.at[t], dst_ref=ag_buf.at[t + 1],
                send_sem=ag_send_sems.at[t], recv_sem=ag_recv_sems.at[t],
                device_id=(right,), device_id_type=pl.DeviceIdType.MESH)
            rdma.start()
            rdma.wait()
            c_in = lax.rem(my_pos + N_DEV - t, N_DEV)
            out_ref[pl.ds(c_in * QBLK, QBLK), :] = (
                ag_buf[t + 1].astype(jnp.float32))

    out = pl.pallas_call(
        body,
        out_shape=jax.ShapeDtypeStruct((Sq, Dout), jnp.float32),
        in_specs=[pl.BlockSpec(memory_space=pltpu.VMEM)] * 5,
        out_specs=pl.BlockSpec(memory_space=pltpu.VMEM),
        scratch_shapes=[
            pltpu.VMEM((SQ, dloc), jnp.bfloat16),
            pltpu.VMEM((N_DEV - 1, QBLK, Dout), jnp.bfloat16),
            pltpu.VMEM((N_DEV - 1, QBLK, Dout), jnp.bfloat16),
            pltpu.VMEM((N_DEV, QBLK, Dout), jnp.bfloat16),
            pltpu.SemaphoreType.DMA((N_DEV - 1,)),
            pltpu.SemaphoreType.DMA((N_DEV - 1,)),
            pltpu.SemaphoreType.DMA((N_DEV - 1,)),
            pltpu.SemaphoreType.DMA((N_DEV - 1,)),
        ],
        compiler_params=pltpu.CompilerParams(
            collective_id=0, vmem_limit_bytes=100 * 1024 * 1024),
    )(x2, Wq_my, K2, V2, Wo_my)
    return out[None]


# device time: 139894 ns/iter; 2.0126x vs baseline; 1.0759x over previous
import jax
import jax.numpy as jnp
from jax import lax
from jax.experimental import pallas as pl
from jax.experimental.pallas import tpu as pltpu

N_DEV = 4
SQ = 2048
SKV = 2048
DH = 128
SCALE = 0.08838834764831843
QBLK = 512
N_CHUNK = SQ // QBLK


def kernel(x, Wq, K_ext, V_ext, Wo):
    my = lax.axis_index("i")
    _, Sq, Dm = x.shape
    _, Skv, Hloc, Dh = K_ext.shape
    dloc = Hloc * Dh
    Dout = Wo.shape[1]

    x2 = x[0].astype(jnp.bfloat16)
    Wq_my = lax.dynamic_slice(Wq, (0, my * dloc), (Dm, dloc)).astype(jnp.bfloat16)
    K2 = K_ext[0].reshape(Skv, dloc).astype(jnp.bfloat16)
    V2 = V_ext[0].reshape(Skv, dloc).astype(jnp.bfloat16)
    Wo_my = lax.dynamic_slice(Wo, (my * dloc, 0), (dloc, Dout)).astype(jnp.bfloat16)

    def body(x_ref, wq_ref, k_ref, v_ref, wo_ref, out_ref,
             q_scr, ctx_scr, rs_send, rs_recv, ag_buf,
             rs_send_sems, rs_recv_sems, ag_send_sems, ag_recv_sems):
        my_pos = lax.axis_index("i")
        left = lax.rem(my_pos + N_DEV - 1, N_DEV)
        right = lax.rem(my_pos + 1, N_DEV)

        barrier_sem = pltpu.get_barrier_semaphore()
        for nbr in [left, right]:
            pl.semaphore_signal(barrier_sem, inc=1, device_id=(nbr,),
                                device_id_type=pl.DeviceIdType.MESH)
        pl.semaphore_wait(barrier_sem, 2)

        q_scr[...] = lax.dot_general(
            x_ref[...], wq_ref[...], (((1,), (0,)), ((), ())),
            preferred_element_type=jnp.float32).astype(jnp.bfloat16)

        BAND = 896
        GW = 128
        CW = SKV - BAND
        TR = 64

        def dotf32(a, b, dims):
            return lax.dot_general(a, b, (dims, ((), ())),
                                   preferred_element_type=jnp.float32)

        def compute_chunk(c):
            cs = c * QBLK
            bs = pl.multiple_of(jnp.clip(cs - 128, 0, SKV - BAND), 128)
            qblk = q_scr[pl.ds(cs, QBLK), :]
            kB = k_ref[pl.ds(bs, BAND), :]
            vB = v_ref[pl.ds(bs, BAND), :]

            qiA = lax.broadcasted_iota(jnp.int32, (QBLK, GW), 0) + cs
            kiA = lax.broadcasted_iota(jnp.int32, (QBLK, GW), 1)
            mA = (jnp.abs(qiA - kiA) <= 128) | (kiA < 32) | (qiA < 32)
            mA = mA & (kiA < bs)
            qiB = lax.broadcasted_iota(jnp.int32, (QBLK, BAND), 0) + cs
            kiB = lax.broadcasted_iota(jnp.int32, (QBLK, BAND), 1) + bs
            mB = (jnp.abs(qiB - kiB) <= 128) | (kiB < 32) | (qiB < 32)
            qiC = lax.broadcasted_iota(jnp.int32, (TR, CW), 0) + cs
            kiC = lax.broadcasted_iota(jnp.int32, (TR, CW), 1) + BAND
            mC = (qiC < 32) & (kiC >= bs + BAND)

            for h in range(Hloc):
                hd = slice(h * DH, (h + 1) * DH)
                qh = qblk[:, hd]
                sA = dotf32(qh, k_ref[0:GW, hd], ((1,), (1,))) * SCALE
                sB = dotf32(qh, kB[:, hd], ((1,), (1,))) * SCALE
                sC = dotf32(qh[0:TR, :], k_ref[BAND:, hd],
                            ((1,), (1,))) * SCALE
                sA = jnp.where(mA, sA, -1e9)
                sB = jnp.where(mB, sB, -1e9)
                sC = jnp.where(mC, sC, -1e9)

                m_t = jnp.maximum(
                    jnp.maximum(jnp.max(sA[0:TR], axis=1, keepdims=True),
                                jnp.max(sB[0:TR], axis=1, keepdims=True)),
                    jnp.max(sC, axis=1, keepdims=True))
                wA_t = jnp.exp(sA[0:TR] - m_t)
                wB_t = jnp.exp(sB[0:TR] - m_t)
                wC_t = jnp.exp(sC - m_t)
                den_t = (jnp.sum(wA_t, axis=1, keepdims=True)
                         + jnp.sum(wB_t, axis=1, keepdims=True)
                         + jnp.sum(wC_t, axis=1, keepdims=True))
                ctx_t = (dotf32(wA_t.astype(jnp.bfloat16),
                                v_ref[0:GW, hd], ((1,), (0,)))
                         + dotf32(wB_t.astype(jnp.bfloat16),
                                  vB[:, hd], ((1,), (0,)))
                         + dotf32(wC_t.astype(jnp.bfloat16),
                                  v_ref[BAND:, hd], ((1,), (0,)))) / den_t

                m_b = jnp.maximum(
                    jnp.max(sA[TR:], axis=1, keepdims=True),
                    jnp.max(sB[TR:], axis=1, keepdims=True))
                wA_b = jnp.exp(sA[TR:] - m_b)
                wB_b = jnp.exp(sB[TR:] - m_b)
                den_b = (jnp.sum(wA_b, axis=1, keepdims=True)
                         + jnp.sum(wB_b, axis=1, keepdims=True))
                ctx_b = (dotf32(wA_b.astype(jnp.bfloat16),
                                v_ref[0:GW, hd], ((1,), (0,)))
                         + dotf32(wB_b.astype(jnp.bfloat16),
                                  vB[:, hd], ((1,), (0,)))) / den_b

                ctx_scr[0:TR, hd] = ctx_t.astype(jnp.bfloat16)
                ctx_scr[TR:, hd] = ctx_b.astype(jnp.bfloat16)
            return dotf32(ctx_scr[...], wo_ref[...], ((1,), (0,)))

        def rs_rdma(s):
            return pltpu.make_async_remote_copy(
                src_ref=rs_send.at[s], dst_ref=rs_recv.at[s],
                send_sem=rs_send_sems.at[s], recv_sem=rs_recv_sems.at[s],
                device_id=(right,), device_id_type=pl.DeviceIdType.MESH)

        acc = compute_chunk(my_pos)
        rs_send[0] = acc.astype(jnp.bfloat16)
        rdma = rs_rdma(0)
        rdma.start()
        for s in range(1, N_DEV - 1):
            c = lax.rem(my_pos + N_DEV - s, N_DEV)
            acc = compute_chunk(c)
            rdma.wait()
            red = acc + rs_recv[s - 1].astype(jnp.float32)
            rs_send[s] = red.astype(jnp.bfloat16)
            rdma = rs_rdma(s)
            rdma.start()
        c_own = lax.rem(my_pos + 1, N_DEV)
        acc = compute_chunk(c_own)
        rdma.wait()
        final = acc + rs_recv[N_DEV - 2].astype(jnp.float32)
        out_ref[pl.ds(c_own * QBLK, QBLK), :] = final

        ag_buf[0] = final.astype(jnp.bfloat16)
        for t in range(N_DEV - 1):
            rdma = pltpu.make_async_remote_copy(
                src_ref=ag_buf.at[t], dst_ref=ag_buf.at[t + 1],
                send_sem=ag_send_sems.at[t], recv_sem=ag_recv_sems.at[t],
                device_id=(right,), device_id_type=pl.DeviceIdType.MESH)
            rdma.start()
            rdma.wait()
            c_in = lax.rem(my_pos + N_DEV - t, N_DEV)
            out_ref[pl.ds(c_in * QBLK, QBLK), :] = (
                ag_buf[t + 1].astype(jnp.float32))

    out = pl.pallas_call(
        body,
        out_shape=jax.ShapeDtypeStruct((Sq, Dout), jnp.float32),
        in_specs=[pl.BlockSpec(memory_space=pltpu.VMEM)] * 5,
        out_specs=pl.BlockSpec(memory_space=pltpu.VMEM),
        scratch_shapes=[
            pltpu.VMEM((SQ, dloc), jnp.bfloat16),
            pltpu.VMEM((QBLK, dloc), jnp.bfloat16),
            pltpu.VMEM((N_DEV - 1, QBLK, Dout), jnp.bfloat16),
            pltpu.VMEM((N_DEV - 1, QBLK, Dout), jnp.bfloat16),
            pltpu.VMEM((N_DEV, QBLK, Dout), jnp.bfloat16),
            pltpu.SemaphoreType.DMA((N_DEV - 1,)),
            pltpu.SemaphoreType.DMA((N_DEV - 1,)),
            pltpu.SemaphoreType.DMA((N_DEV - 1,)),
            pltpu.SemaphoreType.DMA((N_DEV - 1,)),
        ],
        compiler_params=pltpu.CompilerParams(
            collective_id=0, vmem_limit_bytes=100 * 1024 * 1024),
    )(x2, Wq_my, K2, V2, Wo_my)
    return out[None]


# device time: 137344 ns/iter; 2.0499x vs baseline; 1.0186x over previous
import jax
import jax.numpy as jnp
from jax import lax
from jax.experimental import pallas as pl
from jax.experimental.pallas import tpu as pltpu

N_DEV = 4
SQ = 2048
SKV = 2048
DH = 128
SCALE = 0.08838834764831843
QBLK = 512
N_CHUNK = SQ // QBLK


def kernel(x, Wq, K_ext, V_ext, Wo):
    my = lax.axis_index("i")
    _, Sq, Dm = x.shape
    _, Skv, Hloc, Dh = K_ext.shape
    dloc = Hloc * Dh
    Dout = Wo.shape[1]

    x2 = x[0].astype(jnp.bfloat16)
    Wq_my = lax.dynamic_slice(Wq, (0, my * dloc), (Dm, dloc)).astype(jnp.bfloat16)
    K2 = K_ext[0].reshape(Skv, dloc).astype(jnp.bfloat16)
    V2 = V_ext[0].reshape(Skv, dloc).astype(jnp.bfloat16)
    Wo_my = lax.dynamic_slice(Wo, (my * dloc, 0), (dloc, Dout)).astype(jnp.bfloat16)

    def body(x_ref, wq_ref, k_ref, v_ref, wo_ref, out_ref,
             q_scr, ctx_scr, rs_send, rs_recv, ag_buf,
             rs_send_sems, rs_recv_sems, ag_send_sems, ag_recv_sems):
        my_pos = lax.axis_index("i")
        left = lax.rem(my_pos + N_DEV - 1, N_DEV)
        right = lax.rem(my_pos + 1, N_DEV)

        barrier_sem = pltpu.get_barrier_semaphore()
        for nbr in [left, right]:
            pl.semaphore_signal(barrier_sem, inc=1, device_id=(nbr,),
                                device_id_type=pl.DeviceIdType.MESH)
        pl.semaphore_wait(barrier_sem, 2)

        q_scr[...] = (lax.dot_general(
            x_ref[...], wq_ref[...], (((1,), (0,)), ((), ())),
            preferred_element_type=jnp.float32) * SCALE).astype(jnp.bfloat16)

        BAND = 896
        GW = 128
        CW = SKV - BAND
        TR = 64

        def dotf32(a, b, dims):
            return lax.dot_general(a, b, (dims, ((), ())),
                                   preferred_element_type=jnp.float32)

        def compute_chunk(c):
            cs = c * QBLK
            bs = pl.multiple_of(jnp.clip(cs - 128, 0, SKV - BAND), 128)
            qblk = q_scr[pl.ds(cs, QBLK), :]
            kB = k_ref[pl.ds(bs, BAND), :]
            vB = v_ref[pl.ds(bs, BAND), :]

            qiA = lax.broadcasted_iota(jnp.int32, (QBLK, GW), 0) + cs
            kiA = lax.broadcasted_iota(jnp.int32, (QBLK, GW), 1)
            mA = (jnp.abs(qiA - kiA) <= 128) | (kiA < 32) | (qiA < 32)
            mA = mA & (kiA < bs)
            qiB = lax.broadcasted_iota(jnp.int32, (QBLK, BAND), 0) + cs
            kiB = lax.broadcasted_iota(jnp.int32, (QBLK, BAND), 1) + bs
            mB = (jnp.abs(qiB - kiB) <= 128) | (kiB < 32) | (qiB < 32)
            qiC = lax.broadcasted_iota(jnp.int32, (TR, CW), 0) + cs
            kiC = lax.broadcasted_iota(jnp.int32, (TR, CW), 1) + BAND
            mC = (qiC < 32) & (kiC >= bs + BAND)

            for h in range(Hloc):
                hd = slice(h * DH, (h + 1) * DH)
                qh = qblk[:, hd]
                sA = dotf32(qh, k_ref[0:GW, hd], ((1,), (1,)))
                sB = dotf32(qh, kB[:, hd], ((1,), (1,)))
                sC = dotf32(qh[0:TR, :], k_ref[BAND:, hd], ((1,), (1,)))
                sA = jnp.where(mA, sA, -1e9)
                sB = jnp.where(mB, sB, -1e9)
                sC = jnp.where(mC, sC, -1e9)

                wA_t = jnp.exp(sA[0:TR])
                wB_t = jnp.exp(sB[0:TR])
                wC_t = jnp.exp(sC)
                den_t = (jnp.sum(wA_t, axis=1, keepdims=True)
                         + jnp.sum(wB_t, axis=1, keepdims=True)
                         + jnp.sum(wC_t, axis=1, keepdims=True))
                ctx_t = (dotf32(wA_t.astype(jnp.bfloat16),
                                v_ref[0:GW, hd], ((1,), (0,)))
                         + dotf32(wB_t.astype(jnp.bfloat16),
                                  vB[:, hd], ((1,), (0,)))
                         + dotf32(wC_t.astype(jnp.bfloat16),
                                  v_ref[BAND:, hd], ((1,), (0,)))) / den_t

                wA_b = jnp.exp(sA[TR:])
                wB_b = jnp.exp(sB[TR:])
                den_b = (jnp.sum(wA_b, axis=1, keepdims=True)
                         + jnp.sum(wB_b, axis=1, keepdims=True))
                ctx_b = (dotf32(wA_b.astype(jnp.bfloat16),
                                v_ref[0:GW, hd], ((1,), (0,)))
                         + dotf32(wB_b.astype(jnp.bfloat16),
                                  vB[:, hd], ((1,), (0,)))) / den_b

                ctx_scr[0:TR, hd] = ctx_t.astype(jnp.bfloat16)
                ctx_scr[TR:, hd] = ctx_b.astype(jnp.bfloat16)
            return dotf32(ctx_scr[...], wo_ref[...], ((1,), (0,)))

        def rs_rdma(s):
            return pltpu.make_async_remote_copy(
                src_ref=rs_send.at[s], dst_ref=rs_recv.at[s],
                send_sem=rs_send_sems.at[s], recv_sem=rs_recv_sems.at[s],
                device_id=(right,), device_id_type=pl.DeviceIdType.MESH)

        acc = compute_chunk(my_pos)
        rs_send[0] = acc.astype(jnp.bfloat16)
        rdma = rs_rdma(0)
        rdma.start()
        for s in range(1, N_DEV - 1):
            c = lax.rem(my_pos + N_DEV - s, N_DEV)
            acc = compute_chunk(c)
            rdma.wait()
            red = acc + rs_recv[s - 1].astype(jnp.float32)
            rs_send[s] = red.astype(jnp.bfloat16)
            rdma = rs_rdma(s)
            rdma.start()
        c_own = lax.rem(my_pos + 1, N_DEV)
        acc = compute_chunk(c_own)
        rdma.wait()
        final = acc + rs_recv[N_DEV - 2].astype(jnp.float32)
        out_ref[pl.ds(c_own * QBLK, QBLK), :] = final

        ag_buf[0] = final.astype(jnp.bfloat16)
        for t in range(N_DEV - 1):
            rdma = pltpu.make_async_remote_copy(
                src_ref=ag_buf.at[t], dst_ref=ag_buf.at[t + 1],
                send_sem=ag_send_sems.at[t], recv_sem=ag_recv_sems.at[t],
                device_id=(right,), device_id_type=pl.DeviceIdType.MESH)
            rdma.start()
            rdma.wait()
            c_in = lax.rem(my_pos + N_DEV - t, N_DEV)
            out_ref[pl.ds(c_in * QBLK, QBLK), :] = (
                ag_buf[t + 1].astype(jnp.float32))

    out = pl.pallas_call(
        body,
        out_shape=jax.ShapeDtypeStruct((Sq, Dout), jnp.float32),
        in_specs=[pl.BlockSpec(memory_space=pltpu.VMEM)] * 5,
        out_specs=pl.BlockSpec(memory_space=pltpu.VMEM),
        scratch_shapes=[
            pltpu.VMEM((SQ, dloc), jnp.bfloat16),
            pltpu.VMEM((QBLK, dloc), jnp.bfloat16),
            pltpu.VMEM((N_DEV - 1, QBLK, Dout), jnp.bfloat16),
            pltpu.VMEM((N_DEV - 1, QBLK, Dout), jnp.bfloat16),
            pltpu.VMEM((N_DEV, QBLK, Dout), jnp.bfloat16),
            pltpu.SemaphoreType.DMA((N_DEV - 1,)),
            pltpu.SemaphoreType.DMA((N_DEV - 1,)),
            pltpu.SemaphoreType.DMA((N_DEV - 1,)),
            pltpu.SemaphoreType.DMA((N_DEV - 1,)),
        ],
        compiler_params=pltpu.CompilerParams(
            collective_id=0, vmem_limit_bytes=100 * 1024 * 1024),
    )(x2, Wq_my, K2, V2, Wo_my)
    return out[None]


# device time: 124575 ns/iter; 2.2601x vs baseline; 1.1025x over previous
import jax
import jax.numpy as jnp
from jax import lax
from jax.experimental import pallas as pl
from jax.experimental.pallas import tpu as pltpu

N_DEV = 4
SQ = 2048
SKV = 2048
DH = 128
SCALE = 0.08838834764831843
QBLK = 512
N_CHUNK = SQ // QBLK


def kernel(x, Wq, K_ext, V_ext, Wo):
    my = lax.axis_index("i")
    _, Sq, Dm = x.shape
    _, Skv, Hloc, Dh = K_ext.shape
    dloc = Hloc * Dh
    Dout = Wo.shape[1]

    x2 = x[0].astype(jnp.bfloat16)
    Wq_my = lax.dynamic_slice(Wq, (0, my * dloc), (Dm, dloc)).astype(jnp.bfloat16)
    K2 = K_ext[0].reshape(Skv, dloc).astype(jnp.bfloat16)
    V2 = V_ext[0].reshape(Skv, dloc).astype(jnp.bfloat16)
    Wo_my = lax.dynamic_slice(Wo, (my * dloc, 0), (dloc, Dout)).astype(jnp.bfloat16)

    def body(x_ref, wq_ref, k_ref, v_ref, wo_ref, out_ref,
             q_scr, ctx_scr, rs_send, rs_recv, ag_buf,
             rs_send_sems, rs_recv_sems, ag_send_sems, ag_recv_sems):
        my_pos = lax.axis_index("i")
        left = lax.rem(my_pos + N_DEV - 1, N_DEV)
        right = lax.rem(my_pos + 1, N_DEV)

        barrier_sem = pltpu.get_barrier_semaphore()
        for nbr in [left, right]:
            pl.semaphore_signal(barrier_sem, inc=1, device_id=(nbr,),
                                device_id_type=pl.DeviceIdType.MESH)
        pl.semaphore_wait(barrier_sem, 2)

        q_scr[...] = (lax.dot_general(
            x_ref[...], wq_ref[...], (((1,), (0,)), ((), ())),
            preferred_element_type=jnp.float32) * SCALE).astype(jnp.bfloat16)

        BAND = 896
        GW = 128
        CW = SKV - BAND
        TR = 64

        def dotf32(a, b, dims):
            return lax.dot_general(a, b, (dims, ((), ())),
                                   preferred_element_type=jnp.float32)

        def compute_chunk(c):
            cs = c * QBLK
            bs = pl.multiple_of(jnp.clip(cs - 128, 0, SKV - BAND), 128)
            qblk = q_scr[pl.ds(cs, QBLK), :]
            kB = k_ref[pl.ds(bs, BAND), :]
            vB = v_ref[pl.ds(bs, BAND), :]

            qiA = lax.broadcasted_iota(jnp.int32, (QBLK, GW), 0) + cs
            kiA = lax.broadcasted_iota(jnp.int32, (QBLK, GW), 1)
            mA = (jnp.abs(qiA - kiA) <= 128) | (kiA < 32) | (qiA < 32)
            mA = mA & (kiA < bs)
            qiB = lax.broadcasted_iota(jnp.int32, (QBLK, BAND), 0) + cs
            kiB = lax.broadcasted_iota(jnp.int32, (QBLK, BAND), 1) + bs
            mB = (jnp.abs(qiB - kiB) <= 128) | (kiB < 32) | (qiB < 32)
            qiC = lax.broadcasted_iota(jnp.int32, (TR, CW), 0) + cs
            kiC = lax.broadcasted_iota(jnp.int32, (TR, CW), 1) + BAND
            mC = (qiC < 32) & (kiC >= bs + BAND)

            for h in range(Hloc):
                hd = slice(h * DH, (h + 1) * DH)
                qh = qblk[:, hd]
                sA = dotf32(qh, k_ref[0:GW, hd], ((1,), (1,)))
                sB = dotf32(qh, kB[:, hd], ((1,), (1,)))
                sC = dotf32(qh[0:TR, :], k_ref[BAND:, hd], ((1,), (1,)))
                sA = jnp.where(mA, sA, -1e9)
                sB = jnp.where(mB, sB, -1e9)
                sC = jnp.where(mC, sC, -1e9)

                wA_t = jnp.exp(sA[0:TR])
                wB_t = jnp.exp(sB[0:TR])
                wC_t = jnp.exp(sC)
                den_t = (jnp.sum(wA_t, axis=1, keepdims=True)
                         + jnp.sum(wB_t, axis=1, keepdims=True)
                         + jnp.sum(wC_t, axis=1, keepdims=True))
                ctx_t = (dotf32(wA_t.astype(jnp.bfloat16),
                                v_ref[0:GW, hd], ((1,), (0,)))
                         + dotf32(wB_t.astype(jnp.bfloat16),
                                  vB[:, hd], ((1,), (0,)))
                         + dotf32(wC_t.astype(jnp.bfloat16),
                                  v_ref[BAND:, hd], ((1,), (0,)))) / den_t

                wA_b = jnp.exp(sA[TR:])
                wB_b = jnp.exp(sB[TR:])
                den_b = (jnp.sum(wA_b, axis=1, keepdims=True)
                         + jnp.sum(wB_b, axis=1, keepdims=True))
                ctx_b = (dotf32(wA_b.astype(jnp.bfloat16),
                                v_ref[0:GW, hd], ((1,), (0,)))
                         + dotf32(wB_b.astype(jnp.bfloat16),
                                  vB[:, hd], ((1,), (0,)))) / den_b

                ctx_scr[0:TR, hd] = ctx_t.astype(jnp.bfloat16)
                ctx_scr[TR:, hd] = ctx_b.astype(jnp.bfloat16)
            return dotf32(ctx_scr[...], wo_ref[...], ((1,), (0,)))

        def rs_rdma(s):
            return pltpu.make_async_remote_copy(
                src_ref=rs_send.at[s], dst_ref=rs_recv.at[s],
                send_sem=rs_send_sems.at[s], recv_sem=rs_recv_sems.at[s],
                device_id=(right,), device_id_type=pl.DeviceIdType.MESH)

        acc = compute_chunk(my_pos)
        rs_send[0] = acc.astype(jnp.bfloat16)
        rdma = rs_rdma(0)
        rdma.start()
        for s in range(1, N_DEV - 1):
            c = lax.rem(my_pos + N_DEV - s, N_DEV)
            acc = compute_chunk(c)
            rdma.wait()
            red = acc + rs_recv[s - 1].astype(jnp.float32)
            rs_send[s] = red.astype(jnp.bfloat16)
            rdma = rs_rdma(s)
            rdma.start()
        c_own = lax.rem(my_pos + 1, N_DEV)
        acc = compute_chunk(c_own)
        rdma.wait()
        final = acc + rs_recv[N_DEV - 2].astype(jnp.float32)
        out_ref[pl.ds(c_own * QBLK, QBLK), :] = final

        ag_buf[0] = final.astype(jnp.bfloat16)
        rdma_r = pltpu.make_async_remote_copy(
            src_ref=ag_buf.at[0], dst_ref=ag_buf.at[1],
            send_sem=ag_send_sems.at[0], recv_sem=ag_recv_sems.at[0],
            device_id=(right,), device_id_type=pl.DeviceIdType.MESH)
        rdma_l = pltpu.make_async_remote_copy(
            src_ref=ag_buf.at[0], dst_ref=ag_buf.at[2],
            send_sem=ag_send_sems.at[1], recv_sem=ag_recv_sems.at[1],
            device_id=(left,), device_id_type=pl.DeviceIdType.MESH)
        rdma_r.start()
        rdma_l.start()
        rdma_r.wait_recv()
        rdma_f = pltpu.make_async_remote_copy(
            src_ref=ag_buf.at[1], dst_ref=ag_buf.at[3],
            send_sem=ag_send_sems.at[2], recv_sem=ag_recv_sems.at[2],
            device_id=(right,), device_id_type=pl.DeviceIdType.MESH)
        rdma_f.start()
        out_ref[pl.ds(my_pos * QBLK, QBLK), :] = ag_buf[1].astype(jnp.float32)
        rdma_l.wait_recv()
        c_r = lax.rem(my_pos + 2, N_DEV)
        out_ref[pl.ds(c_r * QBLK, QBLK), :] = ag_buf[2].astype(jnp.float32)
        rdma_f.wait_recv()
        c_f = lax.rem(my_pos + 3, N_DEV)
        out_ref[pl.ds(c_f * QBLK, QBLK), :] = ag_buf[3].astype(jnp.float32)
        rdma_r.wait_send()
        rdma_l.wait_send()
        rdma_f.wait_send()

    out = pl.pallas_call(
        body,
        out_shape=jax.ShapeDtypeStruct((Sq, Dout), jnp.float32),
        in_specs=[pl.BlockSpec(memory_space=pltpu.VMEM)] * 5,
        out_specs=pl.BlockSpec(memory_space=pltpu.VMEM),
        scratch_shapes=[
            pltpu.VMEM((SQ, dloc), jnp.bfloat16),
            pltpu.VMEM((QBLK, dloc), jnp.bfloat16),
            pltpu.VMEM((N_DEV - 1, QBLK, Dout), jnp.bfloat16),
            pltpu.VMEM((N_DEV - 1, QBLK, Dout), jnp.bfloat16),
            pltpu.VMEM((N_DEV, QBLK, Dout), jnp.bfloat16),
            pltpu.SemaphoreType.DMA((N_DEV - 1,)),
            pltpu.SemaphoreType.DMA((N_DEV - 1,)),
            pltpu.SemaphoreType.DMA((N_DEV - 1,)),
            pltpu.SemaphoreType.DMA((N_DEV - 1,)),
        ],
        compiler_params=pltpu.CompilerParams(
            collective_id=0, vmem_limit_bytes=100 * 1024 * 1024),
    )(x2, Wq_my, K2, V2, Wo_my)
    return out[None]


# device time: 122316 ns/iter; 2.3018x vs baseline; 1.0185x over previous
import jax
import jax.numpy as jnp
from jax import lax
from jax.experimental import pallas as pl
from jax.experimental.pallas import tpu as pltpu

N_DEV = 4
SQ = 2048
SKV = 2048
DH = 128
SCALE = 0.08838834764831843
QBLK = 512
N_CHUNK = SQ // QBLK


def kernel(x, Wq, K_ext, V_ext, Wo):
    my = lax.axis_index("i")
    _, Sq, Dm = x.shape
    _, Skv, Hloc, Dh = K_ext.shape
    dloc = Hloc * Dh
    Dout = Wo.shape[1]

    x2 = x[0]
    Wq_my = lax.dynamic_slice(Wq, (0, my * dloc), (Dm, dloc))
    K2 = K_ext[0].reshape(Skv, dloc).astype(jnp.bfloat16)
    V2 = V_ext[0].reshape(Skv, dloc).astype(jnp.bfloat16)
    Wo_my = lax.dynamic_slice(Wo, (my * dloc, 0), (dloc, Dout))

    def body(x_ref, wq_ref, k_ref, v_ref, wo_f32, out_ref,
             q_scr, ctx_scr, wo_ref, rs_send, rs_recv, ag_buf,
             rs_send_sems, rs_recv_sems, ag_send_sems, ag_recv_sems):
        my_pos = lax.axis_index("i")
        left = lax.rem(my_pos + N_DEV - 1, N_DEV)
        right = lax.rem(my_pos + 1, N_DEV)

        barrier_sem = pltpu.get_barrier_semaphore()
        for nbr in [left, right]:
            pl.semaphore_signal(barrier_sem, inc=1, device_id=(nbr,),
                                device_id_type=pl.DeviceIdType.MESH)
        pl.semaphore_wait(barrier_sem, 2)

        wo_ref[...] = wo_f32[...].astype(jnp.bfloat16)
        q_scr[...] = (lax.dot_general(
            x_ref[...].astype(jnp.bfloat16), wq_ref[...].astype(jnp.bfloat16),
            (((1,), (0,)), ((), ())),
            preferred_element_type=jnp.float32) * SCALE).astype(jnp.bfloat16)

        BAND = 896
        GW = 128
        CW = SKV - BAND
        TR = 64

        def dotf32(a, b, dims):
            return lax.dot_general(a, b, (dims, ((), ())),
                                   preferred_element_type=jnp.float32)

        def compute_chunk(c):
            cs = c * QBLK
            bs = pl.multiple_of(jnp.clip(cs - 128, 0, SKV - BAND), 128)
            qblk = q_scr[pl.ds(cs, QBLK), :]
            kB = k_ref[pl.ds(bs, BAND), :]
            vB = v_ref[pl.ds(bs, BAND), :]

            qiA = lax.broadcasted_iota(jnp.int32, (QBLK, GW), 0) + cs
            kiA = lax.broadcasted_iota(jnp.int32, (QBLK, GW), 1)
            mA = (jnp.abs(qiA - kiA) <= 128) | (kiA < 32) | (qiA < 32)
            mA = mA & (kiA < bs)
            qiB = lax.broadcasted_iota(jnp.int32, (QBLK, BAND), 0) + cs
            kiB = lax.broadcasted_iota(jnp.int32, (QBLK, BAND), 1) + bs
            mB = (jnp.abs(qiB - kiB) <= 128) | (kiB < 32) | (qiB < 32)
            qiC = lax.broadcasted_iota(jnp.int32, (TR, CW), 0) + cs
            kiC = lax.broadcasted_iota(jnp.int32, (TR, CW), 1) + BAND
            mC = (qiC < 32) & (kiC >= bs + BAND)

            for h in range(Hloc):
                hd = slice(h * DH, (h + 1) * DH)
                qh = qblk[:, hd]
                sA = dotf32(qh, k_ref[0:GW, hd], ((1,), (1,)))
                sB = dotf32(qh, kB[:, hd], ((1,), (1,)))
                sC = dotf32(qh[0:TR, :], k_ref[BAND:, hd], ((1,), (1,)))
                sA = jnp.where(mA, sA, -1e9)
                sB = jnp.where(mB, sB, -1e9)
                sC = jnp.where(mC, sC, -1e9)

                wA_t = jnp.exp(sA[0:TR])
                wB_t = jnp.exp(sB[0:TR])
                wC_t = jnp.exp(sC)
                den_t = (jnp.sum(wA_t, axis=1, keepdims=True)
                         + jnp.sum(wB_t, axis=1, keepdims=True)
                         + jnp.sum(wC_t, axis=1, keepdims=True))
                ctx_t = (dotf32(wA_t.astype(jnp.bfloat16),
                                v_ref[0:GW, hd], ((1,), (0,)))
                         + dotf32(wB_t.astype(jnp.bfloat16),
                                  vB[:, hd], ((1,), (0,)))
                         + dotf32(wC_t.astype(jnp.bfloat16),
                                  v_ref[BAND:, hd], ((1,), (0,)))) / den_t

                wA_b = jnp.exp(sA[TR:])
                wB_b = jnp.exp(sB[TR:])
                den_b = (jnp.sum(wA_b, axis=1, keepdims=True)
                         + jnp.sum(wB_b, axis=1, keepdims=True))
                ctx_b = (dotf32(wA_b.astype(jnp.bfloat16),
                                v_ref[0:GW, hd], ((1,), (0,)))
                         + dotf32(wB_b.astype(jnp.bfloat16),
                                  vB[:, hd], ((1,), (0,)))) / den_b

                ctx_scr[0:TR, hd] = ctx_t.astype(jnp.bfloat16)
                ctx_scr[TR:, hd] = ctx_b.astype(jnp.bfloat16)
            return dotf32(ctx_scr[...], wo_ref[...], ((1,), (0,)))

        def rs_rdma(s):
            return pltpu.make_async_remote_copy(
                src_ref=rs_send.at[s], dst_ref=rs_recv.at[s],
                send_sem=rs_send_sems.at[s], recv_sem=rs_recv_sems.at[s],
                device_id=(right,), device_id_type=pl.DeviceIdType.MESH)

        acc = compute_chunk(my_pos)
        rs_send[0] = acc.astype(jnp.bfloat16)
        rdma = rs_rdma(0)
        rdma.start()
        for s in range(1, N_DEV - 1):
            c = lax.rem(my_pos + N_DEV - s, N_DEV)
            acc = compute_chunk(c)
            rdma.wait()
            red = acc + rs_recv[s - 1].astype(jnp.float32)
            rs_send[s] = red.astype(jnp.bfloat16)
            rdma = rs_rdma(s)
            rdma.start()
        c_own = lax.rem(my_pos + 1, N_DEV)
        acc = compute_chunk(c_own)
        rdma.wait()
        final = acc + rs_recv[N_DEV - 2].astype(jnp.float32)
        out_ref[pl.ds(c_own * QBLK, QBLK), :] = final.astype(jnp.bfloat16)

        ag_buf[0] = final.astype(jnp.bfloat16)
        rdma_r = pltpu.make_async_remote_copy(
            src_ref=ag_buf.at[0], dst_ref=ag_buf.at[1],
            send_sem=ag_send_sems.at[0], recv_sem=ag_recv_sems.at[0],
            device_id=(right,), device_id_type=pl.DeviceIdType.MESH)
        rdma_l = pltpu.make_async_remote_copy(
            src_ref=ag_buf.at[0], dst_ref=ag_buf.at[2],
            send_sem=ag_send_sems.at[1], recv_sem=ag_recv_sems.at[1],
            device_id=(left,), device_id_type=pl.DeviceIdType.MESH)
        rdma_r.start()
        rdma_l.start()
        rdma_r.wait_recv()
        rdma_f = pltpu.make_async_remote_copy(
            src_ref=ag_buf.at[1], dst_ref=ag_buf.at[3],
            send_sem=ag_send_sems.at[2], recv_sem=ag_recv_sems.at[2],
            device_id=(right,), device_id_type=pl.DeviceIdType.MESH)
        rdma_f.start()
        out_ref[pl.ds(my_pos * QBLK, QBLK), :] = ag_buf[1]
        rdma_l.wait_recv()
        c_r = lax.rem(my_pos + 2, N_DEV)
        out_ref[pl.ds(c_r * QBLK, QBLK), :] = ag_buf[2]
        rdma_f.wait_recv()
        c_f = lax.rem(my_pos + 3, N_DEV)
        out_ref[pl.ds(c_f * QBLK, QBLK), :] = ag_buf[3]
        rdma_r.wait_send()
        rdma_l.wait_send()
        rdma_f.wait_send()

    out = pl.pallas_call(
        body,
        out_shape=jax.ShapeDtypeStruct((Sq, Dout), jnp.bfloat16),
        in_specs=[pl.BlockSpec(memory_space=pltpu.VMEM)] * 5,
        out_specs=pl.BlockSpec(memory_space=pltpu.VMEM),
        scratch_shapes=[
            pltpu.VMEM((SQ, dloc), jnp.bfloat16),
            pltpu.VMEM((QBLK, dloc), jnp.bfloat16),
            pltpu.VMEM((dloc, Dout), jnp.bfloat16),
            pltpu.VMEM((N_DEV - 1, QBLK, Dout), jnp.bfloat16),
            pltpu.VMEM((N_DEV - 1, QBLK, Dout), jnp.bfloat16),
            pltpu.VMEM((N_DEV, QBLK, Dout), jnp.bfloat16),
            pltpu.SemaphoreType.DMA((N_DEV - 1,)),
            pltpu.SemaphoreType.DMA((N_DEV - 1,)),
            pltpu.SemaphoreType.DMA((N_DEV - 1,)),
            pltpu.SemaphoreType.DMA((N_DEV - 1,)),
        ],
        compiler_params=pltpu.CompilerParams(
            collective_id=0, vmem_limit_bytes=100 * 1024 * 1024),
    )(x2, Wq_my, K2, V2, Wo_my)
    return out[None]


# device time: 121282 ns/iter; 2.3214x vs baseline; 1.0085x over previous
import jax
import jax.numpy as jnp
from jax import lax
from jax.experimental import pallas as pl
from jax.experimental.pallas import tpu as pltpu

N_DEV = 4
SQ = 2048
SKV = 2048
DH = 128
SCALE = 0.08838834764831843
QBLK = 512
N_CHUNK = SQ // QBLK


def kernel(x, Wq, K_ext, V_ext, Wo):
    my = lax.axis_index("i")
    _, Sq, Dm = x.shape
    _, Skv, Hloc, Dh = K_ext.shape
    dloc = Hloc * Dh
    Dout = Wo.shape[1]

    x2 = x[0]
    Wq_my = lax.dynamic_slice(Wq, (0, my * dloc), (Dm, dloc))
    K2 = K_ext[0].reshape(Skv, dloc).astype(jnp.bfloat16)
    V2 = V_ext[0].reshape(Skv, dloc).astype(jnp.bfloat16)
    Wo_my = lax.dynamic_slice(Wo, (my * dloc, 0), (dloc, Dout))

    def body(x_ref, wq_ref, k_ref, v_ref, wo_f32, out_ref,
             q_scr, ctx_scr, wo_ref, rs_send, rs_recv, ag_buf,
             rs_send_sems, rs_recv_sems, ag_send_sems, ag_recv_sems):
        my_pos = lax.axis_index("i")
        left = lax.rem(my_pos + N_DEV - 1, N_DEV)
        right = lax.rem(my_pos + 1, N_DEV)

        barrier_sem = pltpu.get_barrier_semaphore()
        for nbr in [left, right]:
            pl.semaphore_signal(barrier_sem, inc=1, device_id=(nbr,),
                                device_id_type=pl.DeviceIdType.MESH)
        pl.semaphore_wait(barrier_sem, 2)

        wo_ref[...] = wo_f32[...].astype(jnp.bfloat16)
        q_scr[...] = (lax.dot_general(
            x_ref[...].astype(jnp.bfloat16), wq_ref[...].astype(jnp.bfloat16),
            (((1,), (0,)), ((), ())),
            preferred_element_type=jnp.float32) * SCALE).astype(jnp.bfloat16)

        BAND = 768
        GW = 128
        CW = SKV - BAND
        TR = 64

        def dotf32(a, b, dims):
            return lax.dot_general(a, b, (dims, ((), ())),
                                   preferred_element_type=jnp.float32)

        def compute_chunk(c):
            cs = c * QBLK
            bs = pl.multiple_of(jnp.clip(cs - 128, 0, SKV - BAND), 128)
            qblk = q_scr[pl.ds(cs, QBLK), :]
            kB = k_ref[pl.ds(bs, BAND), :]
            vB = v_ref[pl.ds(bs, BAND), :]

            qiA = lax.broadcasted_iota(jnp.int32, (QBLK, GW), 0) + cs
            kiA = lax.broadcasted_iota(jnp.int32, (QBLK, GW), 1)
            mA = (jnp.abs(qiA - kiA) <= 128) | (kiA < 32) | (qiA < 32)
            mA = mA & (kiA < bs)
            qiB = lax.broadcasted_iota(jnp.int32, (QBLK, BAND), 0) + cs
            kiB = lax.broadcasted_iota(jnp.int32, (QBLK, BAND), 1) + bs
            mB = (jnp.abs(qiB - kiB) <= 128) | (kiB < 32) | (qiB < 32)

            for h in range(Hloc):
                hd = slice(h * DH, (h + 1) * DH)
                qh = qblk[:, hd]
                sA = jnp.where(mA, dotf32(qh, k_ref[0:GW, hd],
                                          ((1,), (1,))), -1e9)
                sB = jnp.where(mB, dotf32(qh, kB[:, hd],
                                          ((1,), (1,))), -1e9)
                wA = jnp.exp(sA)
                wB = jnp.exp(sB)
                den = (jnp.sum(wA, axis=1, keepdims=True)
                       + jnp.sum(wB, axis=1, keepdims=True))
                ctx = (dotf32(wA.astype(jnp.bfloat16),
                              v_ref[0:GW, hd], ((1,), (0,)))
                       + dotf32(wB.astype(jnp.bfloat16),
                                vB[:, hd], ((1,), (0,)))) / den
                ctx_scr[:, hd] = ctx.astype(jnp.bfloat16)

            @pl.when(c == 0)
            def _():
                qi = lax.broadcasted_iota(jnp.int32, (TR, BAND), 0)
                ki = lax.broadcasted_iota(jnp.int32, (TR, BAND), 1)
                mBt = (jnp.abs(qi - ki) <= 128) | (ki < 32) | (qi < 32)
                mCt = lax.broadcasted_iota(jnp.int32, (TR, CW), 0) < 32
                for h in range(Hloc):
                    hd = slice(h * DH, (h + 1) * DH)
                    qt = q_scr[0:TR, hd]
                    sB = jnp.where(mBt, dotf32(qt, k_ref[0:BAND, hd],
                                               ((1,), (1,))), -1e9)
                    sC = jnp.where(mCt, dotf32(qt, k_ref[BAND:, hd],
                                               ((1,), (1,))), -1e9)
                    wB = jnp.exp(sB)
                    wC = jnp.exp(sC)
                    den = (jnp.sum(wB, axis=1, keepdims=True)
                           + jnp.sum(wC, axis=1, keepdims=True))
                    ctx = (dotf32(wB.astype(jnp.bfloat16),
                                  v_ref[0:BAND, hd], ((1,), (0,)))
                           + dotf32(wC.astype(jnp.bfloat16),
                                    v_ref[BAND:, hd], ((1,), (0,)))) / den
                    ctx_scr[0:TR, hd] = ctx.astype(jnp.bfloat16)

            return dotf32(ctx_scr[...], wo_ref[...], ((1,), (0,)))

        def rs_rdma(s):
            return pltpu.make_async_remote_copy(
                src_ref=rs_send.at[s], dst_ref=rs_recv.at[s],
                send_sem=rs_send_sems.at[s], recv_sem=rs_recv_sems.at[s],
                device_id=(right,), device_id_type=pl.DeviceIdType.MESH)

        acc = compute_chunk(my_pos)
        rs_send[0] = acc.astype(jnp.bfloat16)
        rdma = rs_rdma(0)
        rdma.start()
        for s in range(1, N_DEV - 1):
            c = lax.rem(my_pos + N_DEV - s, N_DEV)
            acc = compute_chunk(c)
            rdma.wait()
            red = acc + rs_recv[s - 1].astype(jnp.float32)
            rs_send[s] = red.astype(jnp.bfloat16)
            rdma = rs_rdma(s)
            rdma.start()
        c_own = lax.rem(my_pos + 1, N_DEV)
        acc = compute_chunk(c_own)
        rdma.wait()
        final = acc + rs_recv[N_DEV - 2].astype(jnp.float32)
        out_ref[pl.ds(c_own * QBLK, QBLK), :] = final.astype(jnp.bfloat16)

        ag_buf[0] = final.astype(jnp.bfloat16)
        rdma_r = pltpu.make_async_remote_copy(
            src_ref=ag_buf.at[0], dst_ref=ag_buf.at[1],
            send_sem=ag_send_sems.at[0], recv_sem=ag_recv_sems.at[0],
            device_id=(right,), device_id_type=pl.DeviceIdType.MESH)
        rdma_l = pltpu.make_async_remote_copy(
            src_ref=ag_buf.at[0], dst_ref=ag_buf.at[2],
            send_sem=ag_send_sems.at[1], recv_sem=ag_recv_sems.at[1],
            device_id=(left,), device_id_type=pl.DeviceIdType.MESH)
        rdma_r.start()
        rdma_l.start()
        rdma_r.wait_recv()
        rdma_f = pltpu.make_async_remote_copy(
            src_ref=ag_buf.at[1], dst_ref=ag_buf.at[3],
            send_sem=ag_send_sems.at[2], recv_sem=ag_recv_sems.at[2],
            device_id=(right,), device_id_type=pl.DeviceIdType.MESH)
        rdma_f.start()
        out_ref[pl.ds(my_pos * QBLK, QBLK), :] = ag_buf[1]
        rdma_l.wait_recv()
        c_r = lax.rem(my_pos + 2, N_DEV)
        out_ref[pl.ds(c_r * QBLK, QBLK), :] = ag_buf[2]
        rdma_f.wait_recv()
        c_f = lax.rem(my_pos + 3, N_DEV)
        out_ref[pl.ds(c_f * QBLK, QBLK), :] = ag_buf[3]
        rdma_r.wait_send()
        rdma_l.wait_send()
        rdma_f.wait_send()

    out = pl.pallas_call(
        body,
        out_shape=jax.ShapeDtypeStruct((Sq, Dout), jnp.bfloat16),
        in_specs=[pl.BlockSpec(memory_space=pltpu.VMEM)] * 5,
        out_specs=pl.BlockSpec(memory_space=pltpu.VMEM),
        scratch_shapes=[
            pltpu.VMEM((SQ, dloc), jnp.bfloat16),
            pltpu.VMEM((QBLK, dloc), jnp.bfloat16),
            pltpu.VMEM((dloc, Dout), jnp.bfloat16),
            pltpu.VMEM((N_DEV - 1, QBLK, Dout), jnp.bfloat16),
            pltpu.VMEM((N_DEV - 1, QBLK, Dout), jnp.bfloat16),
            pltpu.VMEM((N_DEV, QBLK, Dout), jnp.bfloat16),
            pltpu.SemaphoreType.DMA((N_DEV - 1,)),
            pltpu.SemaphoreType.DMA((N_DEV - 1,)),
            pltpu.SemaphoreType.DMA((N_DEV - 1,)),
            pltpu.SemaphoreType.DMA((N_DEV - 1,)),
        ],
        compiler_params=pltpu.CompilerParams(
            collective_id=0, vmem_limit_bytes=100 * 1024 * 1024),
    )(x2, Wq_my, K2, V2, Wo_my)
    return out[None]


# device time: 115981 ns/iter; 2.4275x vs baseline; 1.0457x over previous
import jax
import jax.numpy as jnp
from jax import lax
from jax.experimental import pallas as pl
from jax.experimental.pallas import tpu as pltpu

N_DEV = 4
SQ = 2048
SKV = 2048
DH = 128
SCALE = 0.08838834764831843
QBLK = 512
N_CHUNK = SQ // QBLK


def kernel(x, Wq, K_ext, V_ext, Wo):
    my = lax.axis_index("i")
    _, Sq, Dm = x.shape
    _, Skv, Hloc, Dh = K_ext.shape
    dloc = Hloc * Dh
    Dout = Wo.shape[1]

    x2 = x[0]
    Wq_my = lax.dynamic_slice(Wq, (0, my * dloc), (Dm, dloc))
    K2 = K_ext[0].reshape(Skv, dloc).astype(jnp.bfloat16)
    V2 = V_ext[0].reshape(Skv, dloc).astype(jnp.bfloat16)
    Wo_my = lax.dynamic_slice(Wo, (my * dloc, 0), (dloc, Dout))

    def body(x_ref, wq_ref, k_ref, v_ref, wo_f32, out_ref,
             q_scr, ctx_scr, wo_ref, rs_send, rs_recv, ag_buf,
             rs_send_sems, rs_recv_sems, ag_send_sems, ag_recv_sems):
        my_pos = lax.axis_index("i")
        left = lax.rem(my_pos + N_DEV - 1, N_DEV)
        right = lax.rem(my_pos + 1, N_DEV)

        barrier_sem = pltpu.get_barrier_semaphore()
        for nbr in [left, right]:
            pl.semaphore_signal(barrier_sem, inc=1, device_id=(nbr,),
                                device_id_type=pl.DeviceIdType.MESH)
        pl.semaphore_wait(barrier_sem, 2)

        wo_ref[...] = wo_f32[...].astype(jnp.bfloat16)
        q_scr[...] = (lax.dot_general(
            x_ref[...].astype(jnp.bfloat16), wq_ref[...].astype(jnp.bfloat16),
            (((1,), (0,)), ((), ())),
            preferred_element_type=jnp.float32) * SCALE).astype(jnp.bfloat16)

        BAND = 768
        GW = 128
        CW = SKV - BAND
        TR = 64

        def dotf32(a, b, dims):
            return lax.dot_general(a, b, (dims, ((), ())),
                                   preferred_element_type=jnp.float32)

        def compute_chunk(c):
            cs = c * QBLK
            bs = pl.multiple_of(jnp.clip(cs - 128, 0, SKV - BAND), 128)
            qblk = q_scr[pl.ds(cs, QBLK), :]
            kB = k_ref[pl.ds(bs, BAND), :]
            vB = v_ref[pl.ds(bs, BAND), :]

            qiA = lax.broadcasted_iota(jnp.int32, (QBLK, GW), 0) + cs
            kiA = lax.broadcasted_iota(jnp.int32, (QBLK, GW), 1)
            mA = (jnp.abs(qiA - kiA) <= 128) | (kiA < 32) | (qiA < 32)
            mA = mA & (kiA < bs)
            qiB = lax.broadcasted_iota(jnp.int32, (QBLK, BAND), 0) + cs
            kiB = lax.broadcasted_iota(jnp.int32, (QBLK, BAND), 1) + bs
            mB = (jnp.abs(qiB - kiB) <= 128) | (kiB < 32) | (qiB < 32)

            for h in range(Hloc):
                hd = slice(h * DH, (h + 1) * DH)
                qh = qblk[:, hd]
                sA = jnp.where(mA, dotf32(qh, k_ref[0:GW, hd],
                                          ((1,), (1,))), -1e9)
                sB = jnp.where(mB, dotf32(qh, kB[:, hd],
                                          ((1,), (1,))), -1e9)
                wA = jnp.exp(sA)
                wB = jnp.exp(sB)
                den = (jnp.sum(wA, axis=1, keepdims=True)
                       + jnp.sum(wB, axis=1, keepdims=True))
                ctx = (dotf32(wA.astype(jnp.bfloat16),
                              v_ref[0:GW, hd], ((1,), (0,)))
                       + dotf32(wB.astype(jnp.bfloat16),
                                vB[:, hd], ((1,), (0,)))) / den
                ctx_scr[:, hd] = ctx.astype(jnp.bfloat16)

            @pl.when(c == 0)
            def _():
                qi = lax.broadcasted_iota(jnp.int32, (TR, BAND), 0)
                ki = lax.broadcasted_iota(jnp.int32, (TR, BAND), 1)
                mBt = (jnp.abs(qi - ki) <= 128) | (ki < 32) | (qi < 32)
                mCt = lax.broadcasted_iota(jnp.int32, (TR, CW), 0) < 32
                for h in range(Hloc):
                    hd = slice(h * DH, (h + 1) * DH)
                    qt = q_scr[0:TR, hd]
                    sB = jnp.where(mBt, dotf32(qt, k_ref[0:BAND, hd],
                                               ((1,), (1,))), -1e9)
                    sC = jnp.where(mCt, dotf32(qt, k_ref[BAND:, hd],
                                               ((1,), (1,))), -1e9)
                    wB = jnp.exp(sB)
                    wC = jnp.exp(sC)
                    den = (jnp.sum(wB, axis=1, keepdims=True)
                           + jnp.sum(wC, axis=1, keepdims=True))
                    ctx = (dotf32(wB.astype(jnp.bfloat16),
                                  v_ref[0:BAND, hd], ((1,), (0,)))
                           + dotf32(wC.astype(jnp.bfloat16),
                                    v_ref[BAND:, hd], ((1,), (0,)))) / den
                    ctx_scr[0:TR, hd] = ctx.astype(jnp.bfloat16)

            return dotf32(ctx_scr[...], wo_ref[...], ((1,), (0,)))

        def rs_rdma(s):
            return pltpu.make_async_remote_copy(
                src_ref=rs_send.at[s], dst_ref=rs_recv.at[s],
                send_sem=rs_send_sems.at[s], recv_sem=rs_recv_sems.at[s],
                device_id=(right,), device_id_type=pl.DeviceIdType.MESH)

        rs_rdmas = []
        acc = compute_chunk(my_pos)
        rs_send[0] = acc.astype(jnp.bfloat16)
        rdma = rs_rdma(0)
        rdma.start()
        rs_rdmas.append(rdma)
        for s in range(1, N_DEV - 1):
            c = lax.rem(my_pos + N_DEV - s, N_DEV)
            acc = compute_chunk(c)
            rdma.wait_recv()
            red = acc + rs_recv[s - 1].astype(jnp.float32)
            rs_send[s] = red.astype(jnp.bfloat16)
            rdma = rs_rdma(s)
            rdma.start()
            rs_rdmas.append(rdma)
        c_own = lax.rem(my_pos + 1, N_DEV)
        acc = compute_chunk(c_own)
        rdma.wait_recv()
        final_bf = (acc + rs_recv[N_DEV - 2].astype(jnp.float32)).astype(
            jnp.bfloat16)
        ag_buf[0] = final_bf
        out_ref[pl.ds(c_own * QBLK, QBLK), :] = final_bf

        HQB = QBLK // 2
        rdma_r = pltpu.make_async_remote_copy(
            src_ref=ag_buf.at[0], dst_ref=ag_buf.at[1],
            send_sem=ag_send_sems.at[0], recv_sem=ag_recv_sems.at[0],
            device_id=(right,), device_id_type=pl.DeviceIdType.MESH)
        rdma_l = pltpu.make_async_remote_copy(
            src_ref=ag_buf.at[0], dst_ref=ag_buf.at[2],
            send_sem=ag_send_sems.at[1], recv_sem=ag_recv_sems.at[1],
            device_id=(left,), device_id_type=pl.DeviceIdType.MESH)
        rdma_r.start()
        rdma_l.start()
        rdma_r.wait_recv()
        fwd_r = pltpu.make_async_remote_copy(
            src_ref=ag_buf.at[1].at[pl.ds(0, HQB)],
            dst_ref=ag_buf.at[3].at[pl.ds(0, HQB)],
            send_sem=ag_send_sems.at[2], recv_sem=ag_recv_sems.at[2],
            device_id=(right,), device_id_type=pl.DeviceIdType.MESH)
        fwd_r.start()
        out_ref[pl.ds(my_pos * QBLK, QBLK), :] = ag_buf[1]
        rdma_l.wait_recv()
        fwd_l = pltpu.make_async_remote_copy(
            src_ref=ag_buf.at[2].at[pl.ds(HQB, HQB)],
            dst_ref=ag_buf.at[3].at[pl.ds(HQB, HQB)],
            send_sem=ag_send_sems.at[3], recv_sem=ag_recv_sems.at[3],
            device_id=(left,), device_id_type=pl.DeviceIdType.MESH)
        fwd_l.start()
        c_r = lax.rem(my_pos + 2, N_DEV)
        out_ref[pl.ds(c_r * QBLK, QBLK), :] = ag_buf[2]
        fwd_r.wait_recv()
        fwd_l.wait_recv()
        c_f = lax.rem(my_pos + 3, N_DEV)
        out_ref[pl.ds(c_f * QBLK, QBLK), :] = ag_buf[3]
        for r in rs_rdmas:
            r.wait_send()
        rdma_r.wait_send()
        rdma_l.wait_send()
        fwd_r.wait_send()
        fwd_l.wait_send()

    out = pl.pallas_call(
        body,
        out_shape=jax.ShapeDtypeStruct((Sq, Dout), jnp.bfloat16),
        in_specs=[pl.BlockSpec(memory_space=pltpu.VMEM)] * 5,
        out_specs=pl.BlockSpec(memory_space=pltpu.VMEM),
        scratch_shapes=[
            pltpu.VMEM((SQ, dloc), jnp.bfloat16),
            pltpu.VMEM((QBLK, dloc), jnp.bfloat16),
            pltpu.VMEM((dloc, Dout), jnp.bfloat16),
            pltpu.VMEM((N_DEV - 1, QBLK, Dout), jnp.bfloat16),
            pltpu.VMEM((N_DEV - 1, QBLK, Dout), jnp.bfloat16),
            pltpu.VMEM((N_DEV, QBLK, Dout), jnp.bfloat16),
            pltpu.SemaphoreType.DMA((N_DEV - 1,)),
            pltpu.SemaphoreType.DMA((N_DEV - 1,)),
            pltpu.SemaphoreType.DMA((4,)),
            pltpu.SemaphoreType.DMA((4,)),
        ],
        compiler_params=pltpu.CompilerParams(
            collective_id=0, vmem_limit_bytes=100 * 1024 * 1024),
    )(x2, Wq_my, K2, V2, Wo_my)
    return out[None]
